# baseline (speedup 1.0000x reference)
import functools

import jax
import jax.numpy as jnp
from jax import lax
from jax.experimental import pallas as pl
from jax.experimental.pallas import tpu as pltpu
from jax.experimental.pallas import tpu_sc as plsc

_D = 256
_B = 16
_N = 1024
_HW = 32
_CHUNK = 64
_NC = 2
_NS = 16
_NW = _NC * _NS
_TOK = _B * _N
_CHUNKS = _TOK // _CHUNK
_CHUNKS_PER_W = _CHUNKS // _NW
_NBUF = 4


def _build_sc_gather():
    mesh = plsc.VectorSubcoreMesh(core_axis_name="c", subcore_axis_name="s")

    @functools.partial(
        pl.kernel,
        mesh=mesh,
        compiler_params=pltpu.CompilerParams(needs_layout_passes=False),
        out_type=jax.ShapeDtypeStruct((_TOK, _D), jnp.float32),
        scratch_types=[
            pltpu.VMEM((_CHUNKS_PER_W, _CHUNK), jnp.int32),
            pltpu.VMEM((_NBUF, _CHUNK, _D), jnp.float32),
            pltpu.SemaphoreType.DMA,
            pltpu.SemaphoreType.DMA,
            pltpu.SemaphoreType.DMA,
            pltpu.SemaphoreType.DMA,
            pltpu.SemaphoreType.DMA,
            pltpu.SemaphoreType.DMA,
            pltpu.SemaphoreType.DMA,
            pltpu.SemaphoreType.DMA,
        ],
    )
    def k(seq_hbm, emb_hbm, out_hbm, idx_v, g_v, sg0, sg1, sg2, sg3, sw0, sw1, sw2, sw3):
        wid = lax.axis_index("s") * _NC + lax.axis_index("c")
        base = wid * _CHUNKS_PER_W
        sg = [sg0, sg1, sg2, sg3]
        sw = [sw0, sw1, sw2, sw3]

        pltpu.sync_copy(seq_hbm.at[pl.ds(base, _CHUNKS_PER_W)], idx_v)

        def start_gather(c):
            return pltpu.async_copy(
                emb_hbm.at[idx_v.at[c]], g_v.at[c % _NBUF], sg[c % _NBUF]
            )

        def start_write(c):
            return pltpu.async_copy(
                g_v.at[c % _NBUF],
                out_hbm.at[pl.ds((base + c) * _CHUNK, _CHUNK)],
                sw[c % _NBUF],
            )

        gathers = [None] * _CHUNKS_PER_W
        writes = [None] * _CHUNKS_PER_W
        for c in range(min(_NBUF - 1, _CHUNKS_PER_W)):
            gathers[c] = start_gather(c)
        for c in range(_CHUNKS_PER_W):
            gathers[c].wait()
            writes[c] = start_write(c)
            nxt = c + _NBUF - 1
            if nxt < _CHUNKS_PER_W:
                prev = nxt - _NBUF
                if prev >= 0:
                    writes[prev].wait()
                gathers[nxt] = start_gather(nxt)
        for c in range(max(0, _CHUNKS_PER_W - _NBUF), _CHUNKS_PER_W):
            if writes[c] is not None:
                writes[c].wait()

    return k


_sc_gather = _build_sc_gather()


def kernel(seq, embedding):
    seq2 = seq.astype(jnp.int32).reshape(_CHUNKS, _CHUNK)
    rows = _sc_gather(seq2, embedding)
    out = rows.reshape(_B, _HW, _HW, _D)
    return jnp.transpose(out, (0, 3, 1, 2))


# speedup vs baseline: 1.0161x; 1.0161x over previous
import functools

import jax
import jax.numpy as jnp
from jax import lax
from jax.experimental import pallas as pl
from jax.experimental.pallas import tpu as pltpu
from jax.experimental.pallas import tpu_sc as plsc

_D = 256
_B = 16
_N = 1024
_HW = 32
_CHUNK = 128
_NC = 2
_NS = 16
_NW = _NC * _NS
_TOK = _B * _N
_CHUNKS = _TOK // _CHUNK
_CHUNKS_PER_W = _CHUNKS // _NW
_NBUF = 3


def _build_sc_gather():
    mesh = plsc.VectorSubcoreMesh(core_axis_name="c", subcore_axis_name="s")

    @functools.partial(
        pl.kernel,
        mesh=mesh,
        compiler_params=pltpu.CompilerParams(needs_layout_passes=False),
        out_type=jax.ShapeDtypeStruct((_TOK, _D), jnp.float32),
        scratch_types=[
            pltpu.VMEM((_CHUNKS_PER_W, _CHUNK), jnp.int32),
            pltpu.VMEM((_NBUF, _CHUNK, _D), jnp.float32),
            pltpu.SemaphoreType.DMA,
            pltpu.SemaphoreType.DMA,
            pltpu.SemaphoreType.DMA,
            pltpu.SemaphoreType.DMA,
            pltpu.SemaphoreType.DMA,
            pltpu.SemaphoreType.DMA,
        ],
    )
    def k(seq_hbm, emb_hbm, out_hbm, idx_v, g_v, sg0, sg1, sg2, sw0, sw1, sw2):
        wid = lax.axis_index("s") * _NC + lax.axis_index("c")
        base = wid * _CHUNKS_PER_W
        sg = [sg0, sg1, sg2]
        sw = [sw0, sw1, sw2]

        pltpu.sync_copy(seq_hbm.at[pl.ds(base, _CHUNKS_PER_W)], idx_v)

        def start_gather(c):
            return pltpu.async_copy(
                emb_hbm.at[idx_v.at[c]], g_v.at[c % _NBUF], sg[c % _NBUF]
            )

        def start_write(c):
            return pltpu.async_copy(
                g_v.at[c % _NBUF],
                out_hbm.at[pl.ds((base + c) * _CHUNK, _CHUNK)],
                sw[c % _NBUF],
            )

        gathers = [None] * _CHUNKS_PER_W
        writes = [None] * _CHUNKS_PER_W
        for c in range(min(_NBUF - 1, _CHUNKS_PER_W)):
            gathers[c] = start_gather(c)
        for c in range(_CHUNKS_PER_W):
            gathers[c].wait()
            writes[c] = start_write(c)
            nxt = c + _NBUF - 1
            if nxt < _CHUNKS_PER_W:
                prev = nxt - _NBUF
                if prev >= 0:
                    writes[prev].wait()
                gathers[nxt] = start_gather(nxt)
        for c in range(max(0, _CHUNKS_PER_W - _NBUF), _CHUNKS_PER_W):
            if writes[c] is not None:
                writes[c].wait()

    return k


_sc_gather = _build_sc_gather()


def kernel(seq, embedding):
    seq2 = seq.astype(jnp.int32).reshape(_CHUNKS, _CHUNK)
    rows = _sc_gather(seq2, embedding)
    out = rows.reshape(_B, _HW, _HW, _D)
    return jnp.transpose(out, (0, 3, 1, 2))
